# baseline (speedup 1.0000x reference)
import functools

import jax
import jax.numpy as jnp
from jax import lax
from jax.experimental import pallas as pl
from jax.experimental.pallas import tpu as pltpu
from jax.experimental.pallas import tpu_sc as plsc

N = 10000
E = 320000
H = 64
NG = 64
NC = 2
NS = 16
NW = NC * NS
EPW = 10240
NGRP = 80
NPAD = 10240
RPT = NPAD // NS
NB = 8
LA = 4

_mesh = plsc.VectorSubcoreMesh(
    core_axis_name="c", subcore_axis_name="s", num_cores=NC, num_subcores=NS)
_sc_params = pltpu.CompilerParams(use_tc_tiling_on_sc=False)


@functools.partial(
    pl.kernel,
    mesh=_mesh,
    compiler_params=_sc_params,
    out_type=jax.ShapeDtypeStruct((NC, NPAD, 16), jnp.float32),
    scratch_types=[
        pltpu.VMEM((NGRP, 128), jnp.int32),
        pltpu.VMEM((128, 16), jnp.float32),
        pltpu.VMEM((128, 16), jnp.float32),
        pltpu.VMEM_SHARED((NPAD, 16), jnp.float32),
    ],
)
def _deg_sc(dst_hbm, zeros_hbm, ones_hbm, out_hbm, didx, onesv, zbuf, acc):
    c = lax.axis_index("c")
    s = lax.axis_index("s")
    wid = s * NC + c
    pltpu.sync_copy(zeros_hbm, zbuf)
    pltpu.sync_copy(ones_hbm, onesv)
    pltpu.sync_copy(dst_hbm.at[wid], didx)
    for k in range(RPT // 128):
        pltpu.sync_copy(zbuf, acc.at[pl.ds(s * RPT + k * 128, 128)])
    plsc.subcore_barrier()

    def body(j, carry):
        pltpu.sync_copy(onesv, acc.at[didx.at[j]], add=True)
        return carry

    lax.fori_loop(0, NGRP, body, 0)
    plsc.subcore_barrier()
    for k in range(RPT // 128):
        pltpu.sync_copy(acc.at[pl.ds(s * RPT + k * 128, 128)], zbuf)
        pltpu.sync_copy(zbuf, out_hbm.at[c, pl.ds(s * RPT + k * 128, 128)])


@functools.partial(
    pl.kernel,
    mesh=_mesh,
    compiler_params=_sc_params,
    out_type=jax.ShapeDtypeStruct((NC, NPAD, H), jnp.float32),
    scratch_types=[
        pltpu.VMEM((NGRP, 128), jnp.int32),
        pltpu.VMEM((NGRP, 128), jnp.int32),
        [pltpu.VMEM((128, H), jnp.float32)] * NB,
        [pltpu.SemaphoreType.DMA] * NB,
        [pltpu.SemaphoreType.DMA] * NB,
        pltpu.VMEM_SHARED((NPAD, H), jnp.float32),
    ],
)
def _gather_scatter_sc(table_hbm, src_hbm, dst_hbm, zeros_hbm, out_hbm,
                       sidx, didx, gb, gsem, ssem, acc):
    c = lax.axis_index("c")
    s = lax.axis_index("s")
    wid = s * NC + c
    pltpu.sync_copy(zeros_hbm, gb[0])
    for k in range(RPT // 128):
        pltpu.sync_copy(gb[0], acc.at[pl.ds(s * RPT + k * 128, 128)])
    pltpu.sync_copy(src_hbm.at[wid], sidx)
    pltpu.sync_copy(dst_hbm.at[wid], didx)
    plsc.subcore_barrier()

    for b in range(LA):
        pltpu.async_copy(table_hbm.at[sidx.at[b]], gb[b], gsem[b])

    def step(t, carry):
        for b in range(NB):
            j = t * NB + b
            bn = (b + LA) % NB
            pltpu.make_async_copy(
                table_hbm.at[sidx.at[j]], gb[b], gsem[b]).wait()
            pltpu.async_copy(gb[b], acc.at[didx.at[j]], ssem[b], add=True)
            jn = j + LA

            @pl.when(jn < NGRP)
            def _():
                @pl.when(jn >= NB)
                def _():
                    pltpu.make_async_copy(
                        gb[bn], acc.at[didx.at[jn - NB]], ssem[bn]).wait()

                pltpu.async_copy(table_hbm.at[sidx.at[jn]], gb[bn], gsem[bn])
        return carry

    lax.fori_loop(0, NGRP // NB, step, 0)
    for b in range(NB):
        pltpu.make_async_copy(
            gb[b], acc.at[didx.at[NGRP - NB + b]], ssem[b]).wait()
    plsc.subcore_barrier()
    for k in range(RPT // 128):
        pltpu.sync_copy(acc.at[pl.ds(s * RPT + k * 128, 128)], gb[0])
        pltpu.sync_copy(gb[0], out_hbm.at[c, pl.ds(s * RPT + k * 128, 128)])


def _tc_lin(x_ref, w_ref, b_ref, o_ref):
    o_ref[...] = (
        jnp.dot(x_ref[...], w_ref[...], preferred_element_type=jnp.float32)
        + b_ref[...]
    )


def _tc_prep(degp_ref, h0_ref, w_ref, dis_ref, hp_ref):
    deg = jnp.sum(degp_ref[...], axis=(0, 2)) * (1.0 / 16.0) + 1.0
    dis = lax.rsqrt(deg)[:, None]
    dis_ref[...] = dis
    hp_ref[...] = (
        jnp.dot(h0_ref[...], w_ref[...], preferred_element_type=jnp.float32)
        * dis[:N]
    )


def _tc_mid(s_ref, hp_ref, dis_ref, b_ref, wn_ref, o_ref):
    sarr = s_ref[...]
    dis = dis_ref[...][:N]
    h = jax.nn.relu(dis * (sarr[0, :N] + sarr[1, :N] + hp_ref[...]) + b_ref[...])
    o_ref[...] = (
        jnp.dot(h, wn_ref[...], preferred_element_type=jnp.float32) * dis
    )


def _tc_final(s_ref, hp_ref, dis_ref, b_ref, ew_ref, eb_ref, batch_ref,
              pw_ref, pb_ref, o_ref):
    sarr = s_ref[...]
    dis = dis_ref[...][:N]
    h = jax.nn.relu(dis * (sarr[0, :N] + sarr[1, :N] + hp_ref[...]) + b_ref[...])
    z = jax.nn.relu(
        jnp.dot(h, ew_ref[...], preferred_element_type=jnp.float32) + eb_ref[...]
    )
    seg = lax.broadcasted_iota(jnp.int32, (N, NG), 1)
    onehot = (batch_ref[...] == seg).astype(jnp.float32)
    g = lax.dot_general(
        onehot, z, (((0,), (0,)), ((), ())),
        preferred_element_type=jnp.float32,
    )
    o_ref[...] = (
        jnp.dot(g, pw_ref[...], preferred_element_type=jnp.float32) + pb_ref[...]
    )


_lin_call = pl.pallas_call(
    _tc_lin, out_shape=jax.ShapeDtypeStruct((N, H), jnp.float32))
_prep_call = pl.pallas_call(
    _tc_prep,
    out_shape=(jax.ShapeDtypeStruct((NPAD, 1), jnp.float32),
               jax.ShapeDtypeStruct((N, H), jnp.float32)))
_mid_call = pl.pallas_call(
    _tc_mid, out_shape=jax.ShapeDtypeStruct((N, H), jnp.float32))
_final_call = pl.pallas_call(
    _tc_final, out_shape=jax.ShapeDtypeStruct((NG, 1), jnp.float32))


def kernel(x, edge_index, batch, lin_W, lin_b, W1, b1, W2, b2, W3, b3,
           emb_W, emb_b, pred_W, pred_b):
    src = edge_index[0].reshape(NW, E // NW)
    dst = edge_index[1].reshape(NW, E // NW)
    pad = EPW - E // NW
    src3 = jnp.pad(src, ((0, 0), (0, pad))).reshape(NW, NGRP, 128)
    dst3 = jnp.pad(dst, ((0, 0), (0, pad)), constant_values=N).reshape(
        NW, NGRP, 128)
    zeros16 = jnp.zeros((128, 16), jnp.float32)
    ones16 = jnp.ones((128, 16), jnp.float32)
    zeros64 = jnp.zeros((128, H), jnp.float32)

    degp = _deg_sc(dst3, zeros16, ones16)
    h0 = _lin_call(x, lin_W, lin_b.reshape(1, H))
    dis, hp = _prep_call(degp, h0, W1)
    for Wn, b in ((W2, b1), (W3, b2)):
        s_part = _gather_scatter_sc(hp, src3, dst3, zeros64)
        hp = _mid_call(s_part, hp, dis, b.reshape(1, H), Wn)
    s_part = _gather_scatter_sc(hp, src3, dst3, zeros64)
    out = _final_call(s_part, hp, dis, b3.reshape(1, H), emb_W,
                      emb_b.reshape(1, H), batch.reshape(N, 1),
                      pred_W, pred_b.reshape(1, 1))
    return out.reshape(-1)


# speedup vs baseline: 1.8773x; 1.8773x over previous
import functools

import jax
import jax.numpy as jnp
from jax import lax
from jax.experimental import pallas as pl
from jax.experimental.pallas import tpu as pltpu
from jax.experimental.pallas import tpu_sc as plsc

N = 10000
E = 320000
H = 64
NG = 64
NC = 2
NS = 16
NW = NC * NS
EPW = 10240
NGRP = 80
NPAD = 10240
RPT = NPAD // NS
NB = 8
LA = 4
HW = 32
NGRP2 = 160

_mesh = plsc.VectorSubcoreMesh(
    core_axis_name="c", subcore_axis_name="s", num_cores=NC, num_subcores=NS)
_sc_params = pltpu.CompilerParams(use_tc_tiling_on_sc=False)


@functools.partial(
    pl.kernel,
    mesh=_mesh,
    compiler_params=_sc_params,
    out_type=jax.ShapeDtypeStruct((NC, NPAD, 16), jnp.float32),
    scratch_types=[
        pltpu.VMEM((NGRP, 128), jnp.int32),
        pltpu.VMEM((128, 16), jnp.float32),
        pltpu.VMEM((128, 16), jnp.float32),
        pltpu.VMEM_SHARED((NPAD, 16), jnp.float32),
    ],
)
def _deg_sc(dst_hbm, zeros_hbm, ones_hbm, out_hbm, didx, onesv, zbuf, acc):
    c = lax.axis_index("c")
    s = lax.axis_index("s")
    wid = s * NC + c
    pltpu.sync_copy(zeros_hbm, zbuf)
    pltpu.sync_copy(ones_hbm, onesv)
    pltpu.sync_copy(dst_hbm.at[wid], didx)
    for k in range(RPT // 128):
        pltpu.sync_copy(zbuf, acc.at[pl.ds(s * RPT + k * 128, 128)])
    plsc.subcore_barrier()

    def body(j, carry):
        pltpu.sync_copy(onesv, acc.at[didx.at[j]], add=True)
        return carry

    lax.fori_loop(0, NGRP, body, 0)
    plsc.subcore_barrier()
    for k in range(RPT // 128):
        pltpu.sync_copy(acc.at[pl.ds(s * RPT + k * 128, 128)], zbuf)
        pltpu.sync_copy(zbuf, out_hbm.at[c, pl.ds(s * RPT + k * 128, 128)])


@functools.partial(
    pl.kernel,
    mesh=_mesh,
    compiler_params=_sc_params,
    out_type=jax.ShapeDtypeStruct((NC, NPAD, HW), jnp.float32),
    scratch_types=[
        pltpu.VMEM((NGRP2, 128), jnp.int32),
        pltpu.VMEM((NGRP2, 128), jnp.int32),
        [pltpu.VMEM((128, HW), jnp.float32)] * NB,
        [pltpu.SemaphoreType.DMA] * NB,
        [pltpu.SemaphoreType.DMA] * NB,
        pltpu.VMEM_SHARED((NPAD, HW), jnp.float32),
        pltpu.VMEM_SHARED((NPAD, HW), jnp.float32),
    ],
)
def _gather_scatter_sc(table_hbm, src_hbm, dst_hbm, zeros_hbm, out_hbm,
                       sidx, didx, gb, gsem, ssem, acc, tbl):
    c = lax.axis_index("c")
    s = lax.axis_index("s")
    pltpu.sync_copy(zeros_hbm, gb[0])
    for k in range(RPT // 128):
        pltpu.sync_copy(gb[0], acc.at[pl.ds(s * RPT + k * 128, 128)])
        pltpu.sync_copy(table_hbm.at[c, pl.ds(s * RPT + k * 128, 128)], gb[1])
        pltpu.sync_copy(gb[1], tbl.at[pl.ds(s * RPT + k * 128, 128)])
    pltpu.sync_copy(src_hbm.at[s], sidx)
    pltpu.sync_copy(dst_hbm.at[s], didx)
    plsc.subcore_barrier()

    for b in range(LA):
        pltpu.async_copy(tbl.at[sidx.at[b]], gb[b], gsem[b])

    def step(t, carry):
        for b in range(NB):
            j = t * NB + b
            bn = (b + LA) % NB
            pltpu.make_async_copy(
                tbl.at[sidx.at[j]], gb[b], gsem[b]).wait()
            pltpu.async_copy(gb[b], acc.at[didx.at[j]], ssem[b], add=True)
            jn = j + LA

            @pl.when(jn < NGRP2)
            def _():
                @pl.when(jn >= NB)
                def _():
                    pltpu.make_async_copy(
                        gb[bn], acc.at[didx.at[jn - NB]], ssem[bn]).wait()

                pltpu.async_copy(tbl.at[sidx.at[jn]], gb[bn], gsem[bn])
        return carry

    lax.fori_loop(0, NGRP2 // NB, step, 0)
    for b in range(NB):
        pltpu.make_async_copy(
            gb[b], acc.at[didx.at[NGRP2 - NB + b]], ssem[b]).wait()
    plsc.subcore_barrier()
    for k in range(RPT // 128):
        pltpu.sync_copy(acc.at[pl.ds(s * RPT + k * 128, 128)], gb[0])
        pltpu.sync_copy(gb[0], out_hbm.at[c, pl.ds(s * RPT + k * 128, 128)])


def _tc_lin(x_ref, w_ref, b_ref, o_ref):
    o_ref[...] = (
        jnp.dot(x_ref[...], w_ref[...], preferred_element_type=jnp.float32)
        + b_ref[...]
    )


def _split_store(o_ref, val):
    zpad = jnp.zeros((NPAD - N, HW), jnp.float32)
    o_ref[0, pl.ds(0, N), :] = val[:, :HW]
    o_ref[1, pl.ds(0, N), :] = val[:, HW:]
    o_ref[0, pl.ds(N, NPAD - N), :] = zpad
    o_ref[1, pl.ds(N, NPAD - N), :] = zpad


def _tc_prep(degp_ref, h0_ref, w_ref, dis_ref, hp_ref):
    deg = jnp.sum(degp_ref[...], axis=(0, 2)) * (1.0 / 16.0) + 1.0
    dis = lax.rsqrt(deg)[:, None]
    dis_ref[...] = dis
    val = (
        jnp.dot(h0_ref[...], w_ref[...], preferred_element_type=jnp.float32)
        * dis[:N]
    )
    _split_store(hp_ref, val)


def _relu_combine(s_ref, hp_ref, dis, b_ref):
    sarr = s_ref[...]
    hparr = hp_ref[...]
    svs = jnp.concatenate([sarr[0, :N], sarr[1, :N]], axis=-1)
    hpf = jnp.concatenate([hparr[0, :N], hparr[1, :N]], axis=-1)
    return jax.nn.relu(dis * (svs + hpf) + b_ref[...])


def _tc_mid(s_ref, hp_ref, dis_ref, b_ref, wn_ref, o_ref):
    dis = dis_ref[...][:N]
    h = _relu_combine(s_ref, hp_ref, dis, b_ref)
    val = jnp.dot(h, wn_ref[...], preferred_element_type=jnp.float32) * dis
    _split_store(o_ref, val)


def _tc_final(s_ref, hp_ref, dis_ref, b_ref, ew_ref, eb_ref, batch_ref,
              pw_ref, pb_ref, o_ref):
    dis = dis_ref[...][:N]
    h = _relu_combine(s_ref, hp_ref, dis, b_ref)
    z = jax.nn.relu(
        jnp.dot(h, ew_ref[...], preferred_element_type=jnp.float32) + eb_ref[...]
    )
    seg = lax.broadcasted_iota(jnp.int32, (N, NG), 1)
    onehot = (batch_ref[...] == seg).astype(jnp.float32)
    g = lax.dot_general(
        onehot, z, (((0,), (0,)), ((), ())),
        preferred_element_type=jnp.float32,
    )
    o_ref[...] = (
        jnp.dot(g, pw_ref[...], preferred_element_type=jnp.float32) + pb_ref[...]
    )


_lin_call = pl.pallas_call(
    _tc_lin, out_shape=jax.ShapeDtypeStruct((N, H), jnp.float32))
_prep_call = pl.pallas_call(
    _tc_prep,
    out_shape=(jax.ShapeDtypeStruct((NPAD, 1), jnp.float32),
               jax.ShapeDtypeStruct((NC, NPAD, HW), jnp.float32)))
_mid_call = pl.pallas_call(
    _tc_mid, out_shape=jax.ShapeDtypeStruct((NC, NPAD, HW), jnp.float32))
_final_call = pl.pallas_call(
    _tc_final, out_shape=jax.ShapeDtypeStruct((NG, 1), jnp.float32))


def kernel(x, edge_index, batch, lin_W, lin_b, W1, b1, W2, b2, W3, b3,
           emb_W, emb_b, pred_W, pred_b):
    dst = edge_index[1].reshape(NW, E // NW)
    pad = EPW - E // NW
    dst3 = jnp.pad(dst, ((0, 0), (0, pad)), constant_values=N).reshape(
        NW, NGRP, 128)
    src2 = edge_index[0].reshape(NS, E // NS)
    dst2 = edge_index[1].reshape(NS, E // NS)
    pad2 = NGRP2 * 128 - E // NS
    src3b = jnp.pad(src2, ((0, 0), (0, pad2))).reshape(NS, NGRP2, 128)
    dst3b = jnp.pad(dst2, ((0, 0), (0, pad2)), constant_values=N).reshape(
        NS, NGRP2, 128)
    zeros16 = jnp.zeros((128, 16), jnp.float32)
    ones16 = jnp.ones((128, 16), jnp.float32)
    zeros32 = jnp.zeros((128, HW), jnp.float32)

    degp = _deg_sc(dst3, zeros16, ones16)
    h0 = _lin_call(x, lin_W, lin_b.reshape(1, H))
    dis, hp = _prep_call(degp, h0, W1)
    for Wn, b in ((W2, b1), (W3, b2)):
        s_part = _gather_scatter_sc(hp, src3b, dst3b, zeros32)
        hp = _mid_call(s_part, hp, dis, b.reshape(1, H), Wn)
    s_part = _gather_scatter_sc(hp, src3b, dst3b, zeros32)
    out = _final_call(s_part, hp, dis, b3.reshape(1, H), emb_W,
                      emb_b.reshape(1, H), batch.reshape(N, 1),
                      pred_W, pred_b.reshape(1, 1))
    return out.reshape(-1)


# speedup vs baseline: 1.8984x; 1.0112x over previous
import functools

import jax
import jax.numpy as jnp
from jax import lax
from jax.experimental import pallas as pl
from jax.experimental.pallas import tpu as pltpu
from jax.experimental.pallas import tpu_sc as plsc

N = 10000
E = 320000
H = 64
NG = 64
NC = 2
NS = 16
NW = NC * NS
EPW = 10240
NGRP = 80
NPAD = 10240
RPT = NPAD // NS
NB = 8
LA = 4
HW = 32
NGRP2 = 160

_mesh = plsc.VectorSubcoreMesh(
    core_axis_name="c", subcore_axis_name="s", num_cores=NC, num_subcores=NS)
_sc_params = pltpu.CompilerParams(use_tc_tiling_on_sc=False)


@functools.partial(
    pl.kernel,
    mesh=_mesh,
    compiler_params=_sc_params,
    out_type=jax.ShapeDtypeStruct((NC, NPAD, 16), jnp.float32),
    scratch_types=[
        pltpu.VMEM((NGRP, 128), jnp.int32),
        pltpu.VMEM((128, 16), jnp.float32),
        pltpu.VMEM((128, 16), jnp.float32),
        pltpu.VMEM_SHARED((NPAD, 16), jnp.float32),
    ],
)
def _deg_sc(dst_hbm, zeros_hbm, ones_hbm, out_hbm, didx, onesv, zbuf, acc):
    c = lax.axis_index("c")
    s = lax.axis_index("s")
    wid = s * NC + c
    pltpu.sync_copy(zeros_hbm, zbuf)
    pltpu.sync_copy(ones_hbm, onesv)
    pltpu.sync_copy(dst_hbm.at[wid], didx)
    for k in range(RPT // 128):
        pltpu.sync_copy(zbuf, acc.at[pl.ds(s * RPT + k * 128, 128)])
    plsc.subcore_barrier()

    def body(j, carry):
        pltpu.sync_copy(onesv, acc.at[didx.at[j]], add=True)
        return carry

    lax.fori_loop(0, NGRP, body, 0)
    plsc.subcore_barrier()
    for k in range(RPT // 128):
        pltpu.sync_copy(acc.at[pl.ds(s * RPT + k * 128, 128)], zbuf)
        pltpu.sync_copy(zbuf, out_hbm.at[c, pl.ds(s * RPT + k * 128, 128)])


@functools.partial(
    pl.kernel,
    mesh=_mesh,
    compiler_params=_sc_params,
    out_type=jax.ShapeDtypeStruct((NC, NPAD, HW), jnp.float32),
    scratch_types=[
        pltpu.VMEM((NGRP2, 128), jnp.int32),
        pltpu.VMEM((NGRP2, 128), jnp.int32),
        [pltpu.VMEM((128, HW), jnp.float32)] * NB,
        [pltpu.SemaphoreType.DMA] * NB,
        [pltpu.SemaphoreType.DMA] * NB,
        pltpu.VMEM_SHARED((NPAD, HW), jnp.float32),
        pltpu.VMEM_SHARED((NPAD, HW), jnp.float32),
    ],
)
def _gather_scatter_sc(table_hbm, src_hbm, dst_hbm, zeros_hbm, out_hbm,
                       sidx, didx, gb, gsem, ssem, acc, tbl):
    c = lax.axis_index("c")
    s = lax.axis_index("s")
    for k in range(RPT // 128):
        sl = pl.ds(s * RPT + k * 128, 128)
        pltpu.async_copy(table_hbm.at[c, sl], tbl.at[sl], gsem[k])
        pltpu.async_copy(zeros_hbm, acc.at[sl], ssem[k])
    pltpu.async_copy(src_hbm.at[s], sidx, gsem[5])
    pltpu.async_copy(dst_hbm.at[s], didx, gsem[6])
    for k in range(RPT // 128):
        sl = pl.ds(s * RPT + k * 128, 128)
        pltpu.make_async_copy(table_hbm.at[c, sl], tbl.at[sl], gsem[k]).wait()
        pltpu.make_async_copy(zeros_hbm, acc.at[sl], ssem[k]).wait()
    pltpu.make_async_copy(src_hbm.at[s], sidx, gsem[5]).wait()
    pltpu.make_async_copy(dst_hbm.at[s], didx, gsem[6]).wait()
    plsc.subcore_barrier()

    for b in range(LA):
        pltpu.async_copy(tbl.at[sidx.at[b]], gb[b], gsem[b])

    def step(t, carry):
        for b in range(NB):
            j = t * NB + b
            bn = (b + LA) % NB
            pltpu.make_async_copy(
                tbl.at[sidx.at[j]], gb[b], gsem[b]).wait()
            pltpu.async_copy(gb[b], acc.at[didx.at[j]], ssem[b], add=True)
            jn = j + LA

            @pl.when(jn < NGRP2)
            def _():
                @pl.when(jn >= NB)
                def _():
                    pltpu.make_async_copy(
                        gb[bn], acc.at[didx.at[jn - NB]], ssem[bn]).wait()

                pltpu.async_copy(tbl.at[sidx.at[jn]], gb[bn], gsem[bn])
        return carry

    lax.fori_loop(0, NGRP2 // NB, step, 0)
    for b in range(NB):
        pltpu.make_async_copy(
            gb[b], acc.at[didx.at[NGRP2 - NB + b]], ssem[b]).wait()
    plsc.subcore_barrier()
    for k in range(RPT // 128):
        sl = pl.ds(s * RPT + k * 128, 128)
        pltpu.async_copy(acc.at[sl], out_hbm.at[c, sl], gsem[k])
    for k in range(RPT // 128):
        sl = pl.ds(s * RPT + k * 128, 128)
        pltpu.make_async_copy(acc.at[sl], out_hbm.at[c, sl], gsem[k]).wait()


def _split_store(o_ref, val):
    zpad = jnp.zeros((NPAD - N, HW), jnp.float32)
    o_ref[0, pl.ds(0, N), :] = val[:, :HW]
    o_ref[1, pl.ds(0, N), :] = val[:, HW:]
    o_ref[0, pl.ds(N, NPAD - N), :] = zpad
    o_ref[1, pl.ds(N, NPAD - N), :] = zpad


def _tc_prep(degp_ref, x_ref, lw_ref, lb_ref, w_ref, dis_ref, hp_ref):
    deg = jnp.sum(degp_ref[...], axis=(0, 2)) * (1.0 / 16.0) + 1.0
    dis = lax.rsqrt(deg)[:, None]
    dis_ref[...] = dis
    h0 = (
        jnp.dot(x_ref[...], lw_ref[...], preferred_element_type=jnp.float32)
        + lb_ref[...]
    )
    val = (
        jnp.dot(h0, w_ref[...], preferred_element_type=jnp.float32)
        * dis[:N]
    )
    _split_store(hp_ref, val)


def _relu_combine(s_ref, hp_ref, dis, b_ref):
    sarr = s_ref[...]
    hparr = hp_ref[...]
    svs = jnp.concatenate([sarr[0, :N], sarr[1, :N]], axis=-1)
    hpf = jnp.concatenate([hparr[0, :N], hparr[1, :N]], axis=-1)
    return jax.nn.relu(dis * (svs + hpf) + b_ref[...])


def _tc_mid(s_ref, hp_ref, dis_ref, b_ref, wn_ref, o_ref):
    dis = dis_ref[...][:N]
    h = _relu_combine(s_ref, hp_ref, dis, b_ref)
    val = jnp.dot(h, wn_ref[...], preferred_element_type=jnp.float32) * dis
    _split_store(o_ref, val)


def _tc_final(s_ref, hp_ref, dis_ref, b_ref, ew_ref, eb_ref, batch_ref,
              pw_ref, pb_ref, o_ref):
    dis = dis_ref[...][:N]
    h = _relu_combine(s_ref, hp_ref, dis, b_ref)
    z = jax.nn.relu(
        jnp.dot(h, ew_ref[...], preferred_element_type=jnp.float32) + eb_ref[...]
    )
    seg = lax.broadcasted_iota(jnp.int32, (N, NG), 1)
    onehot = (batch_ref[...] == seg).astype(jnp.float32)
    g = lax.dot_general(
        onehot, z, (((0,), (0,)), ((), ())),
        preferred_element_type=jnp.float32,
    )
    o_ref[...] = (
        jnp.dot(g, pw_ref[...], preferred_element_type=jnp.float32) + pb_ref[...]
    )


_prep_call = pl.pallas_call(
    _tc_prep,
    out_shape=(jax.ShapeDtypeStruct((NPAD, 1), jnp.float32),
               jax.ShapeDtypeStruct((NC, NPAD, HW), jnp.float32)))
_mid_call = pl.pallas_call(
    _tc_mid, out_shape=jax.ShapeDtypeStruct((NC, NPAD, HW), jnp.float32))
_final_call = pl.pallas_call(
    _tc_final, out_shape=jax.ShapeDtypeStruct((NG, 1), jnp.float32))


def kernel(x, edge_index, batch, lin_W, lin_b, W1, b1, W2, b2, W3, b3,
           emb_W, emb_b, pred_W, pred_b):
    dst = edge_index[1].reshape(NW, E // NW)
    pad = EPW - E // NW
    dst3 = jnp.pad(dst, ((0, 0), (0, pad)), constant_values=N).reshape(
        NW, NGRP, 128)
    src2 = edge_index[0].reshape(NS, E // NS)
    dst2 = edge_index[1].reshape(NS, E // NS)
    pad2 = NGRP2 * 128 - E // NS
    src3b = jnp.pad(src2, ((0, 0), (0, pad2))).reshape(NS, NGRP2, 128)
    dst3b = jnp.pad(dst2, ((0, 0), (0, pad2)), constant_values=N).reshape(
        NS, NGRP2, 128)
    zeros16 = jnp.zeros((128, 16), jnp.float32)
    ones16 = jnp.ones((128, 16), jnp.float32)
    zeros32 = jnp.zeros((128, HW), jnp.float32)

    degp = _deg_sc(dst3, zeros16, ones16)
    dis, hp = _prep_call(degp, x, lin_W, lin_b.reshape(1, H), W1)
    for Wn, b in ((W2, b1), (W3, b2)):
        s_part = _gather_scatter_sc(hp, src3b, dst3b, zeros32)
        hp = _mid_call(s_part, hp, dis, b.reshape(1, H), Wn)
    s_part = _gather_scatter_sc(hp, src3b, dst3b, zeros32)
    out = _final_call(s_part, hp, dis, b3.reshape(1, H), emb_W,
                      emb_b.reshape(1, H), batch.reshape(N, 1),
                      pred_W, pred_b.reshape(1, 1))
    return out.reshape(-1)


# speedup vs baseline: 1.9016x; 1.0017x over previous
import functools

import jax
import jax.numpy as jnp
from jax import lax
from jax.experimental import pallas as pl
from jax.experimental.pallas import tpu as pltpu
from jax.experimental.pallas import tpu_sc as plsc

N = 10000
E = 320000
H = 64
NG = 64
NC = 2
NS = 16
NW = NC * NS
EPW = 10240
NGRP = 80
NPAD = 10240
RPT = NPAD // NS
NB = 8
LA = 4
HW = 32
NGRP2 = 160


_mesh = plsc.VectorSubcoreMesh(
    core_axis_name="c", subcore_axis_name="s", num_cores=NC, num_subcores=NS)
_sc_params = pltpu.CompilerParams(use_tc_tiling_on_sc=False)


@functools.partial(
    pl.kernel,
    mesh=_mesh,
    compiler_params=_sc_params,
    out_type=jax.ShapeDtypeStruct((NC, NPAD, 16), jnp.float32),
    scratch_types=[
        pltpu.VMEM((NGRP, 128), jnp.int32),
        pltpu.VMEM((128, 16), jnp.float32),
        [pltpu.SemaphoreType.DMA] * NB,
        pltpu.VMEM_SHARED((NPAD, 16), jnp.float32),
    ],
)
def _deg_sc(dst_hbm, zeros_hbm, ones_hbm, out_hbm, didx, onesv, sem, acc):
    c = lax.axis_index("c")
    s = lax.axis_index("s")
    wid = s * NC + c
    for k in range(RPT // 128):
        sl = pl.ds(s * RPT + k * 128, 128)
        pltpu.async_copy(zeros_hbm, acc.at[sl], sem[k])
    pltpu.async_copy(ones_hbm, onesv, sem[5])
    pltpu.async_copy(dst_hbm.at[wid], didx, sem[6])
    for k in range(RPT // 128):
        sl = pl.ds(s * RPT + k * 128, 128)
        pltpu.make_async_copy(zeros_hbm, acc.at[sl], sem[k]).wait()
    pltpu.make_async_copy(ones_hbm, onesv, sem[5]).wait()
    pltpu.make_async_copy(dst_hbm.at[wid], didx, sem[6]).wait()
    plsc.subcore_barrier()

    def step(t, carry):
        for b in range(NB):
            j = t * NB + b

            @pl.when(j >= NB)
            def _():
                pltpu.make_async_copy(
                    onesv, acc.at[didx.at[j - NB]], sem[b]).wait()

            pltpu.async_copy(onesv, acc.at[didx.at[j]], sem[b], add=True)
        return carry

    lax.fori_loop(0, NGRP // NB, step, 0)
    for b in range(NB):
        pltpu.make_async_copy(
            onesv, acc.at[didx.at[NGRP - NB + b]], sem[b]).wait()
    plsc.subcore_barrier()
    for k in range(RPT // 128):
        sl = pl.ds(s * RPT + k * 128, 128)
        pltpu.async_copy(acc.at[sl], out_hbm.at[c, sl], sem[k])
    for k in range(RPT // 128):
        sl = pl.ds(s * RPT + k * 128, 128)
        pltpu.make_async_copy(acc.at[sl], out_hbm.at[c, sl], sem[k]).wait()


@functools.partial(
    pl.kernel,
    mesh=_mesh,
    compiler_params=_sc_params,
    out_type=jax.ShapeDtypeStruct((NC, NPAD, HW), jnp.float32),
    scratch_types=[
        pltpu.VMEM((NGRP2, 128), jnp.int32),
        pltpu.VMEM((NGRP2, 128), jnp.int32),
        [pltpu.VMEM((128, HW), jnp.float32)] * NB,
        [pltpu.SemaphoreType.DMA] * NB,
        [pltpu.SemaphoreType.DMA] * NB,
        pltpu.VMEM_SHARED((NPAD, HW), jnp.float32),
        pltpu.VMEM_SHARED((NPAD, HW), jnp.float32),
    ],
)
def _gather_scatter_sc(table_hbm, src_hbm, dst_hbm, zeros_hbm, out_hbm,
                       sidx, didx, gb, gsem, ssem, acc, tbl):
    c = lax.axis_index("c")
    s = lax.axis_index("s")
    for k in range(RPT // 128):
        sl = pl.ds(s * RPT + k * 128, 128)
        pltpu.async_copy(table_hbm.at[c, sl], tbl.at[sl], gsem[k])
        pltpu.async_copy(zeros_hbm, acc.at[sl], ssem[k])
    pltpu.async_copy(src_hbm.at[s], sidx, gsem[5])
    pltpu.async_copy(dst_hbm.at[s], didx, gsem[6])
    for k in range(RPT // 128):
        sl = pl.ds(s * RPT + k * 128, 128)
        pltpu.make_async_copy(table_hbm.at[c, sl], tbl.at[sl], gsem[k]).wait()
        pltpu.make_async_copy(zeros_hbm, acc.at[sl], ssem[k]).wait()
    pltpu.make_async_copy(src_hbm.at[s], sidx, gsem[5]).wait()
    pltpu.make_async_copy(dst_hbm.at[s], didx, gsem[6]).wait()
    plsc.subcore_barrier()

    for b in range(LA):
        pltpu.async_copy(tbl.at[sidx.at[b]], gb[b], gsem[b])

    def step(t, carry):
        for b in range(NB):
            j = t * NB + b
            bn = (b + LA) % NB
            pltpu.make_async_copy(
                tbl.at[sidx.at[j]], gb[b], gsem[b]).wait()
            pltpu.async_copy(gb[b], acc.at[didx.at[j]], ssem[b], add=True)
            jn = j + LA

            @pl.when(jn < NGRP2)
            def _():
                @pl.when(jn >= NB)
                def _():
                    pltpu.make_async_copy(
                        gb[bn], acc.at[didx.at[jn - NB]], ssem[bn]).wait()

                pltpu.async_copy(tbl.at[sidx.at[jn]], gb[bn], gsem[bn])
        return carry

    lax.fori_loop(0, NGRP2 // NB, step, 0)
    for b in range(NB):
        pltpu.make_async_copy(
            gb[b], acc.at[didx.at[NGRP2 - NB + b]], ssem[b]).wait()
    plsc.subcore_barrier()
    for k in range(RPT // 128):
        sl = pl.ds(s * RPT + k * 128, 128)
        pltpu.async_copy(acc.at[sl], out_hbm.at[c, sl], gsem[k])
    for k in range(RPT // 128):
        sl = pl.ds(s * RPT + k * 128, 128)
        pltpu.make_async_copy(acc.at[sl], out_hbm.at[c, sl], gsem[k]).wait()


def _split_store(o_ref, val):
    zpad = jnp.zeros((NPAD - N, HW), jnp.float32)
    o_ref[0, pl.ds(0, N), :] = val[:, :HW]
    o_ref[1, pl.ds(0, N), :] = val[:, HW:]
    o_ref[0, pl.ds(N, NPAD - N), :] = zpad
    o_ref[1, pl.ds(N, NPAD - N), :] = zpad


def _tc_prep(degp_ref, x_ref, lw_ref, lb_ref, w_ref, dis_ref, hp_ref):
    deg = jnp.sum(degp_ref[...], axis=(0, 2)) * (1.0 / 16.0) + 1.0
    dis = lax.rsqrt(deg)[:, None]
    dis_ref[...] = dis
    h0 = (
        jnp.dot(x_ref[...], lw_ref[...], preferred_element_type=jnp.float32)
        + lb_ref[...]
    )
    val = (
        jnp.dot(h0, w_ref[...], preferred_element_type=jnp.float32)
        * dis[:N]
    )
    _split_store(hp_ref, val)


def _relu_combine(s_ref, hp_ref, dis, b_ref):
    sarr = s_ref[...]
    hparr = hp_ref[...]
    svs = jnp.concatenate([sarr[0, :N], sarr[1, :N]], axis=-1)
    hpf = jnp.concatenate([hparr[0, :N], hparr[1, :N]], axis=-1)
    return jax.nn.relu(dis * (svs + hpf) + b_ref[...])


def _tc_mid(s_ref, hp_ref, dis_ref, b_ref, wn_ref, o_ref):
    dis = dis_ref[...][:N]
    h = _relu_combine(s_ref, hp_ref, dis, b_ref)
    val = jnp.dot(h, wn_ref[...], preferred_element_type=jnp.float32) * dis
    _split_store(o_ref, val)


def _tc_final(s_ref, hp_ref, dis_ref, b_ref, ew_ref, eb_ref, batch_ref,
              pw_ref, pb_ref, o_ref):
    dis = dis_ref[...][:N]
    h = _relu_combine(s_ref, hp_ref, dis, b_ref)
    z = jax.nn.relu(
        jnp.dot(h, ew_ref[...], preferred_element_type=jnp.float32) + eb_ref[...]
    )
    seg = lax.broadcasted_iota(jnp.int32, (N, NG), 1)
    onehot = (batch_ref[...] == seg).astype(jnp.float32)
    g = lax.dot_general(
        onehot, z, (((0,), (0,)), ((), ())),
        preferred_element_type=jnp.float32,
    )
    o_ref[...] = (
        jnp.dot(g, pw_ref[...], preferred_element_type=jnp.float32) + pb_ref[...]
    )


_prep_call = pl.pallas_call(
    _tc_prep,
    out_shape=(jax.ShapeDtypeStruct((NPAD, 1), jnp.float32),
               jax.ShapeDtypeStruct((NC, NPAD, HW), jnp.float32)))
_mid_call = pl.pallas_call(
    _tc_mid, out_shape=jax.ShapeDtypeStruct((NC, NPAD, HW), jnp.float32))
_final_call = pl.pallas_call(
    _tc_final, out_shape=jax.ShapeDtypeStruct((NG, 1), jnp.float32))


def kernel(x, edge_index, batch, lin_W, lin_b, W1, b1, W2, b2, W3, b3,
           emb_W, emb_b, pred_W, pred_b):
    dst = edge_index[1].reshape(NW, E // NW)
    pad = EPW - E // NW
    dst3 = jnp.pad(dst, ((0, 0), (0, pad)), constant_values=N).reshape(
        NW, NGRP, 128)
    src2 = edge_index[0].reshape(NS, E // NS)
    dst2 = edge_index[1].reshape(NS, E // NS)
    pad2 = NGRP2 * 128 - E // NS
    src3b = jnp.pad(src2, ((0, 0), (0, pad2))).reshape(NS, NGRP2, 128)
    dst3b = jnp.pad(dst2, ((0, 0), (0, pad2)), constant_values=N).reshape(
        NS, NGRP2, 128)
    zeros16 = jnp.zeros((128, 16), jnp.float32)
    ones16 = jnp.ones((128, 16), jnp.float32)
    zeros32 = jnp.zeros((128, HW), jnp.float32)

    degp = _deg_sc(dst3, zeros16, ones16)
    dis, hp = _prep_call(degp, x, lin_W, lin_b.reshape(1, H), W1)
    for Wn, b in ((W2, b1), (W3, b2)):
        s_part = _gather_scatter_sc(hp, src3b, dst3b, zeros32)
        hp = _mid_call(s_part, hp, dis, b.reshape(1, H), Wn)
    s_part = _gather_scatter_sc(hp, src3b, dst3b, zeros32)
    out = _final_call(s_part, hp, dis, b3.reshape(1, H), emb_W,
                      emb_b.reshape(1, H), batch.reshape(N, 1),
                      pred_W, pred_b.reshape(1, 1))
    return out.reshape(-1)
